# Initial kernel scaffold; baseline (speedup 1.0000x reference)
import jax
import jax.numpy as jnp
from jax.experimental import pallas as pl


def kernel(x, edge_index, Wl0, Wr0, b0, Wl1, Wr1, b1, Wl2, Wr2, b2, Wl3, Wr3, b3, Wl4, Wr4, b4, Wout, bout):
    raise NotImplementedError("write your pallas kernel here")


# speedup vs baseline: 5.9502x; 5.9502x over previous
import functools

import jax
import jax.numpy as jnp
from jax import lax
from jax.experimental import pallas as pl
from jax.experimental.pallas import tpu as pltpu
from jax.experimental.pallas import tpu_sc as plsc

_N = 10000
_E = 320000
_B = 125
_NC, _NS = 2, 16
_TILES = _NC * _NS
_EPT = _E // _TILES
_CH = _EPT // _B
_RPS = _N // _NS
_ZCH = _RPS // _B
_BM = 1000
_GRID = _N // _BM


_DST = 2000


@functools.lru_cache(maxsize=None)
def _sc_agg(W, with_deg=False):
  mesh = plsc.VectorSubcoreMesh(
      core_axis_name="c", subcore_axis_name="s", num_cores=_NC, num_subcores=_NS)

  out_type = [jax.ShapeDtypeStruct((_NC, _N, W), jnp.float32)]
  scratch = [
      pltpu.VMEM((_CH, _B), jnp.int32),
      pltpu.VMEM((_CH, _B), jnp.int32),
      pltpu.VMEM((_B, W), jnp.float32),
      pltpu.VMEM((_B, W), jnp.float32),
      pltpu.VMEM_SHARED((_N, W), jnp.float32),
      pltpu.SemaphoreType.DMA,
  ]
  if with_deg:
    out_type.append(jax.ShapeDtypeStruct((_NC, _N), jnp.float32))
    scratch += [
        pltpu.VMEM((128,), jnp.float32),
        pltpu.VMEM((_DST,), jnp.float32),
        pltpu.VMEM_SHARED((_N,), jnp.float32),
    ]

  @functools.partial(
      pl.kernel,
      out_type=tuple(out_type) if with_deg else out_type[0],
      mesh=mesh,
      scratch_types=scratch,
      compiler_params=pltpu.CompilerParams(use_tc_tiling_on_sc=False),
  )
  def k(src_hbm, dst_hbm, table_hbm, *rest):
    if with_deg:
      (out_hbm, deg_hbm, src_v, dst_v, rows_v, stage_v, acc_sh, sem,
       ones_v, dstage_v, dacc_sh) = rest
    else:
      out_hbm, src_v, dst_v, rows_v, stage_v, acc_sh, sem = rest
    c = lax.axis_index("c")
    s = lax.axis_index("s")
    wid = s * _NC + c
    pltpu.sync_copy(src_hbm.at[pl.ds(wid * _CH, _CH)], src_v)
    pltpu.sync_copy(dst_hbm.at[pl.ds(wid * _CH, _CH)], dst_v)

    zero16 = jnp.zeros((16,), jnp.float32)

    def zrow(i, carry):
      for j in range(W // 16):
        stage_v[i, pl.ds(j * 16, 16)] = zero16
      return carry

    lax.fori_loop(0, _B, zrow, 0)

    def zcp(i, carry):
      pltpu.sync_copy(stage_v, acc_sh.at[pl.ds(s * _RPS + i * _B, _B)])
      return carry

    lax.fori_loop(0, _ZCH, zcp, 0)

    if with_deg:
      one16 = jnp.ones((16,), jnp.float32)

      def fill1(i, carry):
        ones_v[pl.ds(i * 16, 16)] = one16
        dstage_v[pl.ds(i * 16, 16)] = zero16
        return carry

      lax.fori_loop(0, 128 // 16, fill1, 0)

      def fill0(i, carry):
        dstage_v[pl.ds(i * 16, 16)] = zero16
        return carry

      lax.fori_loop(128 // 16, _DST // 16, fill0, 0)

      @pl.when(s < _N // _DST)
      def _():
        pltpu.sync_copy(dstage_v, dacc_sh.at[pl.ds(s * _DST, _DST)])

    plsc.subcore_barrier()

    if with_deg:
      def body(g, carry):
        pltpu.async_copy(table_hbm.at[src_v.at[g]], rows_v, sem).wait()
        pltpu.sync_copy(rows_v, acc_sh.at[dst_v.at[g]], add=True)
        pltpu.sync_copy(ones_v.at[pl.ds(0, _B)], dacc_sh.at[dst_v.at[g]],
                        add=True)
        return carry
    else:
      def body(g, carry):
        pltpu.async_copy(table_hbm.at[src_v.at[g]], rows_v, sem).wait()
        pltpu.sync_copy(rows_v, acc_sh.at[dst_v.at[g]], add=True)
        return carry

    lax.fori_loop(0, _CH, body, 0)
    plsc.subcore_barrier()

    def wcp(i, carry):
      pltpu.sync_copy(acc_sh.at[pl.ds(s * _RPS + i * _B, _B)], stage_v)
      pltpu.sync_copy(stage_v, out_hbm.at[c, pl.ds(s * _RPS + i * _B, _B)])
      return carry

    lax.fori_loop(0, _ZCH, wcp, 0)

    if with_deg:
      @pl.when(s < _N // _DST)
      def _():
        pltpu.sync_copy(dacc_sh.at[pl.ds(s * _DST, _DST)], dstage_v)
        pltpu.sync_copy(dstage_v, deg_hbm.at[c, pl.ds(s * _DST, _DST)])

  return k


def _rows(d):
  return pl.BlockSpec((_BM, d), lambda i: (i, 0))


def _part(d):
  return pl.BlockSpec((_NC, _BM, d), lambda i: (0, i, 0))


def _full(r, c):
  return pl.BlockSpec((r, c), lambda i: (0, 0))


def _out(d):
  return jax.ShapeDtypeStruct((_N, d), jnp.float32)


def _mm_body(x_ref, w_ref, o_ref):
  o_ref[...] = jnp.dot(x_ref[...], w_ref[...], preferred_element_type=jnp.float32)


def _tc_mm(x, w):
  fin, fout = w.shape
  return pl.pallas_call(
      _mm_body,
      grid=(_GRID,),
      in_specs=[_rows(fin), _full(fin, fout)],
      out_specs=_rows(fout),
      out_shape=_out(fout),
  )(x, w)


def _l0_body(degp, p, x, wr, b, wl1, h1_o, y1_o, inv_o):
  deg = degp[0] + degp[1]
  inv = 1.0 / jnp.maximum(deg, 1.0)
  agg = (p[0] + p[1]) * inv
  h1 = jnp.maximum(
      agg + jnp.dot(x[...], wr[...], preferred_element_type=jnp.float32) + b[...], 0.0)
  h1_o[...] = h1
  y1_o[...] = jnp.dot(h1, wl1[...], preferred_element_type=jnp.float32)
  inv_o[...] = inv


def _l1_body(p, inv, h, wr, b, o):
  agg = (p[0] + p[1]) * inv[...]
  o[...] = jnp.maximum(
      agg + jnp.dot(h[...], wr[...], preferred_element_type=jnp.float32) + b[...], 0.0)


def _l2_body(p, inv, h, wl, wr, b, oa, ob):
  agg = (p[0] + p[1]) * inv[...]
  h3 = jnp.maximum(
      jnp.dot(agg, wl[...], preferred_element_type=jnp.float32)
      + jnp.dot(h[...], wr[...], preferred_element_type=jnp.float32) + b[...], 0.0)
  oa[...] = h3[:, :64]
  ob[...] = h3[:, 64:]


def _l3_body(pa, pb, inv, ha, hb, wla, wlb, wra, wrb, b, oa, ob, oc, od):
  iv = inv[...]
  aa = (pa[0] + pa[1]) * iv
  ab = (pb[0] + pb[1]) * iv
  h4 = jnp.maximum(
      jnp.dot(aa, wla[...], preferred_element_type=jnp.float32)
      + jnp.dot(ab, wlb[...], preferred_element_type=jnp.float32)
      + jnp.dot(ha[...], wra[...], preferred_element_type=jnp.float32)
      + jnp.dot(hb[...], wrb[...], preferred_element_type=jnp.float32)
      + b[...], 0.0)
  oa[...] = h4[:, :64]
  ob[...] = h4[:, 64:128]
  oc[...] = h4[:, 128:192]
  od[...] = h4[:, 192:]


def _l4_body(pa, pb, pc, pd, inv, ha, hb, hc, hd,
             wla, wlb, wlc, wld, wra, wrb, wrc, wrd, b, wout, bout, o):
  iv = inv[...]
  h5 = (jnp.dot((pa[0] + pa[1]) * iv, wla[...], preferred_element_type=jnp.float32)
        + jnp.dot((pb[0] + pb[1]) * iv, wlb[...], preferred_element_type=jnp.float32)
        + jnp.dot((pc[0] + pc[1]) * iv, wlc[...], preferred_element_type=jnp.float32)
        + jnp.dot((pd[0] + pd[1]) * iv, wld[...], preferred_element_type=jnp.float32)
        + jnp.dot(ha[...], wra[...], preferred_element_type=jnp.float32)
        + jnp.dot(hb[...], wrb[...], preferred_element_type=jnp.float32)
        + jnp.dot(hc[...], wrc[...], preferred_element_type=jnp.float32)
        + jnp.dot(hd[...], wrd[...], preferred_element_type=jnp.float32)
        + b[...])
  h5 = jnp.maximum(h5, 0.0)
  logits = jnp.dot(h5, wout[...], preferred_element_type=jnp.float32) + bout[...]
  m = jnp.max(logits, axis=-1, keepdims=True)
  e = jnp.exp(logits - m)
  o[...] = e / jnp.sum(e, axis=-1, keepdims=True)


def kernel(x, edge_index, Wl0, Wr0, b0, Wl1, Wr1, b1, Wl2, Wr2, b2,
           Wl3, Wr3, b3, Wl4, Wr4, b4, Wout, bout):
  src2 = edge_index[0].reshape(_E // _B, _B)
  dst2 = edge_index[1].reshape(_E // _B, _B)
  b0r, b1r, b2r, b3r, b4r = (v.reshape(1, -1) for v in (b0, b1, b2, b3, b4))
  boutr = bout.reshape(1, -1)

  y0 = _tc_mm(x, Wl0)
  p0, degp = _sc_agg(64, with_deg=True)(src2, dst2, y0)
  degp = degp.reshape(_NC, _N, 1)
  h1, y1, inv = pl.pallas_call(
      _l0_body,
      grid=(_GRID,),
      in_specs=[_part(1), _part(64), _rows(128), _full(128, 64),
                _full(1, 64), _full(64, 64)],
      out_specs=[_rows(64), _rows(64), _rows(1)],
      out_shape=[_out(64), _out(64), _out(1)],
  )(degp, p0, x, Wr0, b0r, Wl1)

  p1, _ = _sc_agg(64, with_deg=True)(src2, dst2, y1)
  h2 = pl.pallas_call(
      _l1_body,
      grid=(_GRID,),
      in_specs=[_part(64), _rows(1), _rows(64), _full(64, 64), _full(1, 64)],
      out_specs=_rows(64),
      out_shape=_out(64),
  )(p1, inv, h1, Wr1, b1r)

  p2, _ = _sc_agg(64, with_deg=True)(src2, dst2, h2)
  h3a, h3b = pl.pallas_call(
      _l2_body,
      grid=(_GRID,),
      in_specs=[_part(64), _rows(1), _rows(64), _full(64, 128),
                _full(64, 128), _full(1, 128)],
      out_specs=[_rows(64), _rows(64)],
      out_shape=[_out(64), _out(64)],
  )(p2, inv, h2, Wl2, Wr2, b2r)

  p3a, _ = _sc_agg(64, with_deg=True)(src2, dst2, h3a)
  p3b, _ = _sc_agg(64, with_deg=True)(src2, dst2, h3b)
  h4 = pl.pallas_call(
      _l3_body,
      grid=(_GRID,),
      in_specs=[_part(64), _part(64), _rows(1), _rows(64), _rows(64),
                _full(64, 256), _full(64, 256), _full(64, 256),
                _full(64, 256), _full(1, 256)],
      out_specs=[_rows(64)] * 4,
      out_shape=[_out(64)] * 4,
  )(p3a, p3b, inv, h3a, h3b, Wl3[:64], Wl3[64:], Wr3[:64], Wr3[64:], b3r)

  p4 = [_sc_agg(64, with_deg=True)(src2, dst2, hq)[0] for hq in h4]
  out = pl.pallas_call(
      _l4_body,
      grid=(_GRID,),
      in_specs=[_part(64)] * 4 + [_rows(1)] + [_rows(64)] * 4
               + [_full(64, 512)] * 8 + [_full(1, 512), _full(512, 4),
                                         _full(1, 4)],
      out_specs=_rows(4),
      out_shape=jax.ShapeDtypeStruct((_N, 4), jnp.float32),
  )(p4[0], p4[1], p4[2], p4[3], inv, h4[0], h4[1], h4[2], h4[3],
    Wl4[:64], Wl4[64:128], Wl4[128:192], Wl4[192:],
    Wr4[:64], Wr4[64:128], Wr4[128:192], Wr4[192:],
    b4r, Wout, boutr)
  return out


# speedup vs baseline: 9.1480x; 1.5374x over previous
import functools

import jax
import jax.numpy as jnp
from jax import lax
from jax.experimental import pallas as pl
from jax.experimental.pallas import tpu as pltpu
from jax.experimental.pallas import tpu_sc as plsc

_N = 10000
_E = 320000
_B = 125
_NC, _NS = 2, 16
_TILES = _NC * _NS
_EPT = _E // _TILES
_CH = _EPT // _B
_RPS = _N // _NS
_ZCH = _RPS // _B
_BM = 1000
_GRID = _N // _BM


_DST = 2000


@functools.lru_cache(maxsize=None)
def _sc_agg(W, with_deg=False):
  mesh = plsc.VectorSubcoreMesh(
      core_axis_name="c", subcore_axis_name="s", num_cores=_NC, num_subcores=_NS)

  out_type = [jax.ShapeDtypeStruct((_NC, _N, W), jnp.float32)]
  scratch = [
      pltpu.VMEM((_CH, _B), jnp.int32),
      pltpu.VMEM((_CH, _B), jnp.int32),
      pltpu.VMEM((_B, W), jnp.float32),
      pltpu.VMEM((_B, W), jnp.float32),
      pltpu.VMEM((_B, W), jnp.float32),
      pltpu.VMEM_SHARED((_N, W), jnp.float32),
      pltpu.SemaphoreType.DMA,
      pltpu.SemaphoreType.DMA,
  ]
  if with_deg:
    out_type.append(jax.ShapeDtypeStruct((_NC, _N), jnp.float32))
    scratch += [
        pltpu.VMEM((128,), jnp.float32),
        pltpu.VMEM((_DST,), jnp.float32),
        pltpu.VMEM_SHARED((_N,), jnp.float32),
    ]

  @functools.partial(
      pl.kernel,
      out_type=tuple(out_type) if with_deg else out_type[0],
      mesh=mesh,
      scratch_types=scratch,
      compiler_params=pltpu.CompilerParams(use_tc_tiling_on_sc=False),
  )
  def k(src_hbm, dst_hbm, table_hbm, *rest):
    if with_deg:
      (out_hbm, deg_hbm, src_v, dst_v, rows_a, rows_b, stage_v, acc_sh,
       sem_a, sem_b, ones_v, dstage_v, dacc_sh) = rest
    else:
      (out_hbm, src_v, dst_v, rows_a, rows_b, stage_v, acc_sh,
       sem_a, sem_b) = rest
    c = lax.axis_index("c")
    s = lax.axis_index("s")
    wid = s * _NC + c
    pltpu.sync_copy(src_hbm.at[pl.ds(wid * _CH, _CH)], src_v)
    pltpu.sync_copy(dst_hbm.at[pl.ds(wid * _CH, _CH)], dst_v)

    zero16 = jnp.zeros((16,), jnp.float32)

    def zrow(i, carry):
      for j in range(W // 16):
        stage_v[i, pl.ds(j * 16, 16)] = zero16
      return carry

    lax.fori_loop(0, _B, zrow, 0)

    def zcp(i, carry):
      pltpu.sync_copy(stage_v, acc_sh.at[pl.ds(s * _RPS + i * _B, _B)])
      return carry

    lax.fori_loop(0, _ZCH, zcp, 0)

    if with_deg:
      one16 = jnp.ones((16,), jnp.float32)

      def fill1(i, carry):
        ones_v[pl.ds(i * 16, 16)] = one16
        dstage_v[pl.ds(i * 16, 16)] = zero16
        return carry

      lax.fori_loop(0, 128 // 16, fill1, 0)

      def fill0(i, carry):
        dstage_v[pl.ds(i * 16, 16)] = zero16
        return carry

      lax.fori_loop(128 // 16, _DST // 16, fill0, 0)

      @pl.when(s < _N // _DST)
      def _():
        pltpu.sync_copy(dstage_v, dacc_sh.at[pl.ds(s * _DST, _DST)])

    plsc.subcore_barrier()

    def issue(g, buf, sem):
      pltpu.async_copy(table_hbm.at[src_v.at[g]], buf, sem)

    def drain_scatter(g, buf, sem):
      pltpu.make_async_copy(table_hbm.at[src_v.at[g]], buf, sem).wait()
      pltpu.sync_copy(buf, acc_sh.at[dst_v.at[g]], add=True)
      if with_deg:
        pltpu.sync_copy(ones_v.at[pl.ds(0, _B)], dacc_sh.at[dst_v.at[g]],
                        add=True)

    issue(0, rows_a, sem_a)

    def body(i, carry):
      g = 2 * i
      issue(g + 1, rows_b, sem_b)
      drain_scatter(g, rows_a, sem_a)
      issue(g + 2, rows_a, sem_a)
      drain_scatter(g + 1, rows_b, sem_b)
      return carry

    lax.fori_loop(0, _CH // 2 - 1, body, 0)
    issue(_CH - 1, rows_b, sem_b)
    drain_scatter(_CH - 2, rows_a, sem_a)
    drain_scatter(_CH - 1, rows_b, sem_b)
    plsc.subcore_barrier()

    def wcp(i, carry):
      pltpu.sync_copy(acc_sh.at[pl.ds(s * _RPS + i * _B, _B)], stage_v)
      pltpu.sync_copy(stage_v, out_hbm.at[c, pl.ds(s * _RPS + i * _B, _B)])
      return carry

    lax.fori_loop(0, _ZCH, wcp, 0)

    if with_deg:
      @pl.when(s < _N // _DST)
      def _():
        pltpu.sync_copy(dacc_sh.at[pl.ds(s * _DST, _DST)], dstage_v)
        pltpu.sync_copy(dstage_v, deg_hbm.at[c, pl.ds(s * _DST, _DST)])

  return k


def _rows(d):
  return pl.BlockSpec((_BM, d), lambda i: (i, 0))


def _part(d):
  return pl.BlockSpec((_NC, _BM, d), lambda i: (0, i, 0))


def _full(r, c):
  return pl.BlockSpec((r, c), lambda i: (0, 0))


def _out(d):
  return jax.ShapeDtypeStruct((_N, d), jnp.float32)


def _mm_body(x_ref, w_ref, o_ref):
  o_ref[...] = jnp.dot(x_ref[...], w_ref[...], preferred_element_type=jnp.float32)


def _tc_mm(x, w):
  fin, fout = w.shape
  return pl.pallas_call(
      _mm_body,
      grid=(_GRID,),
      in_specs=[_rows(fin), _full(fin, fout)],
      out_specs=_rows(fout),
      out_shape=_out(fout),
  )(x, w)


def _l0_body(degp, p, x, wr, b, wl1, h1_o, y1_o, inv_o):
  deg = degp[0] + degp[1]
  inv = 1.0 / jnp.maximum(deg, 1.0)
  agg = (p[0] + p[1]) * inv
  h1 = jnp.maximum(
      agg + jnp.dot(x[...], wr[...], preferred_element_type=jnp.float32) + b[...], 0.0)
  h1_o[...] = h1
  y1_o[...] = jnp.dot(h1, wl1[...], preferred_element_type=jnp.float32)
  inv_o[...] = inv


def _l1_body(p, inv, h, wr, b, o):
  agg = (p[0] + p[1]) * inv[...]
  o[...] = jnp.maximum(
      agg + jnp.dot(h[...], wr[...], preferred_element_type=jnp.float32) + b[...], 0.0)


def _l2_body(p, inv, h, wl, wr, b, oa, ob):
  agg = (p[0] + p[1]) * inv[...]
  h3 = jnp.maximum(
      jnp.dot(agg, wl[...], preferred_element_type=jnp.float32)
      + jnp.dot(h[...], wr[...], preferred_element_type=jnp.float32) + b[...], 0.0)
  oa[...] = h3[:, :64]
  ob[...] = h3[:, 64:]


def _l3_body(pa, pb, inv, ha, hb, wla, wlb, wra, wrb, b, oa, ob, oc, od):
  iv = inv[...]
  aa = (pa[0] + pa[1]) * iv
  ab = (pb[0] + pb[1]) * iv
  h4 = jnp.maximum(
      jnp.dot(aa, wla[...], preferred_element_type=jnp.float32)
      + jnp.dot(ab, wlb[...], preferred_element_type=jnp.float32)
      + jnp.dot(ha[...], wra[...], preferred_element_type=jnp.float32)
      + jnp.dot(hb[...], wrb[...], preferred_element_type=jnp.float32)
      + b[...], 0.0)
  oa[...] = h4[:, :64]
  ob[...] = h4[:, 64:128]
  oc[...] = h4[:, 128:192]
  od[...] = h4[:, 192:]


def _l4_body(pa, pb, pc, pd, inv, ha, hb, hc, hd,
             wla, wlb, wlc, wld, wra, wrb, wrc, wrd, b, wout, bout, o):
  iv = inv[...]
  h5 = (jnp.dot((pa[0] + pa[1]) * iv, wla[...], preferred_element_type=jnp.float32)
        + jnp.dot((pb[0] + pb[1]) * iv, wlb[...], preferred_element_type=jnp.float32)
        + jnp.dot((pc[0] + pc[1]) * iv, wlc[...], preferred_element_type=jnp.float32)
        + jnp.dot((pd[0] + pd[1]) * iv, wld[...], preferred_element_type=jnp.float32)
        + jnp.dot(ha[...], wra[...], preferred_element_type=jnp.float32)
        + jnp.dot(hb[...], wrb[...], preferred_element_type=jnp.float32)
        + jnp.dot(hc[...], wrc[...], preferred_element_type=jnp.float32)
        + jnp.dot(hd[...], wrd[...], preferred_element_type=jnp.float32)
        + b[...])
  h5 = jnp.maximum(h5, 0.0)
  logits = jnp.dot(h5, wout[...], preferred_element_type=jnp.float32) + bout[...]
  m = jnp.max(logits, axis=-1, keepdims=True)
  e = jnp.exp(logits - m)
  o[...] = e / jnp.sum(e, axis=-1, keepdims=True)


def kernel(x, edge_index, Wl0, Wr0, b0, Wl1, Wr1, b1, Wl2, Wr2, b2,
           Wl3, Wr3, b3, Wl4, Wr4, b4, Wout, bout):
  src2 = edge_index[0].reshape(_E // _B, _B)
  dst2 = edge_index[1].reshape(_E // _B, _B)
  b0r, b1r, b2r, b3r, b4r = (v.reshape(1, -1) for v in (b0, b1, b2, b3, b4))
  boutr = bout.reshape(1, -1)

  y0 = _tc_mm(x, Wl0)
  p0, degp = _sc_agg(64, with_deg=True)(src2, dst2, y0)
  degp = degp.reshape(_NC, _N, 1)
  h1, y1, inv = pl.pallas_call(
      _l0_body,
      grid=(_GRID,),
      in_specs=[_part(1), _part(64), _rows(128), _full(128, 64),
                _full(1, 64), _full(64, 64)],
      out_specs=[_rows(64), _rows(64), _rows(1)],
      out_shape=[_out(64), _out(64), _out(1)],
  )(degp, p0, x, Wr0, b0r, Wl1)

  p1, _ = _sc_agg(64, with_deg=True)(src2, dst2, y1)
  h2 = pl.pallas_call(
      _l1_body,
      grid=(_GRID,),
      in_specs=[_part(64), _rows(1), _rows(64), _full(64, 64), _full(1, 64)],
      out_specs=_rows(64),
      out_shape=_out(64),
  )(p1, inv, h1, Wr1, b1r)

  p2, _ = _sc_agg(64, with_deg=True)(src2, dst2, h2)
  h3a, h3b = pl.pallas_call(
      _l2_body,
      grid=(_GRID,),
      in_specs=[_part(64), _rows(1), _rows(64), _full(64, 128),
                _full(64, 128), _full(1, 128)],
      out_specs=[_rows(64), _rows(64)],
      out_shape=[_out(64), _out(64)],
  )(p2, inv, h2, Wl2, Wr2, b2r)

  p3a, _ = _sc_agg(64, with_deg=True)(src2, dst2, h3a)
  p3b, _ = _sc_agg(64, with_deg=True)(src2, dst2, h3b)
  h4 = pl.pallas_call(
      _l3_body,
      grid=(_GRID,),
      in_specs=[_part(64), _part(64), _rows(1), _rows(64), _rows(64),
                _full(64, 256), _full(64, 256), _full(64, 256),
                _full(64, 256), _full(1, 256)],
      out_specs=[_rows(64)] * 4,
      out_shape=[_out(64)] * 4,
  )(p3a, p3b, inv, h3a, h3b, Wl3[:64], Wl3[64:], Wr3[:64], Wr3[64:], b3r)

  p4 = [_sc_agg(64, with_deg=True)(src2, dst2, hq)[0] for hq in h4]
  out = pl.pallas_call(
      _l4_body,
      grid=(_GRID,),
      in_specs=[_part(64)] * 4 + [_rows(1)] + [_rows(64)] * 4
               + [_full(64, 512)] * 8 + [_full(1, 512), _full(512, 4),
                                         _full(1, 4)],
      out_specs=_rows(4),
      out_shape=jax.ShapeDtypeStruct((_N, 4), jnp.float32),
  )(p4[0], p4[1], p4[2], p4[3], inv, h4[0], h4[1], h4[2], h4[3],
    Wl4[:64], Wl4[64:128], Wl4[128:192], Wl4[192:],
    Wr4[:64], Wr4[64:128], Wr4[128:192], Wr4[192:],
    b4r, Wout, boutr)
  return out


# speedup vs baseline: 9.4071x; 1.0283x over previous
import functools

import jax
import jax.numpy as jnp
from jax import lax
from jax.experimental import pallas as pl
from jax.experimental.pallas import tpu as pltpu
from jax.experimental.pallas import tpu_sc as plsc

_N = 10000
_E = 320000
_B = 125
_NC, _NS = 2, 16
_TILES = _NC * _NS
_EPT = _E // _TILES
_CH = _EPT // _B
_RPS = _N // _NS
_ZCH = _RPS // _B
_BM = 1000
_GRID = _N // _BM


_DST = 2000


@functools.lru_cache(maxsize=None)
def _sc_agg(W):
  mesh = plsc.VectorSubcoreMesh(
      core_axis_name="c", subcore_axis_name="s", num_cores=_NC, num_subcores=_NS)

  out_type = (jax.ShapeDtypeStruct((_NC, _N, W), jnp.float32),
              jax.ShapeDtypeStruct((_NC, _N), jnp.float32))
  scratch = [
      pltpu.VMEM((_CH, _B), jnp.int32),
      pltpu.VMEM((_CH, _B), jnp.int32),
      pltpu.VMEM((_B, W), jnp.float32),
      pltpu.VMEM((_B, W), jnp.float32),
      pltpu.VMEM((_B, W), jnp.float32),
      pltpu.VMEM_SHARED((_N, W), jnp.float32),
      pltpu.SemaphoreType.DMA,
      pltpu.SemaphoreType.DMA,
      pltpu.VMEM((16,), jnp.int32),
      pltpu.VMEM((128,), jnp.float32),
      pltpu.VMEM((_DST,), jnp.float32),
      pltpu.VMEM_SHARED((_N,), jnp.float32),
  ]

  @functools.partial(
      pl.kernel,
      out_type=out_type,
      mesh=mesh,
      scratch_types=scratch,
      compiler_params=pltpu.CompilerParams(
          use_tc_tiling_on_sc=False, needs_layout_passes=False),
  )
  def k(src_hbm, dst_hbm, table_hbm, dflag_hbm, *rest):
    (out_hbm, deg_hbm, src_v, dst_v, rows_a, rows_b, stage_v, acc_sh,
     sem_a, sem_b, dflag_v, ones_v, dstage_v, dacc_sh) = rest
    c = lax.axis_index("c")
    s = lax.axis_index("s")
    wid = s * _NC + c
    pltpu.sync_copy(src_hbm.at[pl.ds(wid * _CH, _CH)], src_v)
    pltpu.sync_copy(dst_hbm.at[pl.ds(wid * _CH, _CH)], dst_v)
    pltpu.sync_copy(dflag_hbm, dflag_v)
    have_deg = jnp.sum(dflag_v[...]) > 0

    zero16 = jnp.zeros((16,), jnp.float32)

    def zrow(i, carry):
      for j in range(W // 16):
        stage_v[i, pl.ds(j * 16, 16)] = zero16
      return carry

    lax.fori_loop(0, _B, zrow, 0)

    def zcp(i, carry):
      pltpu.sync_copy(stage_v, acc_sh.at[pl.ds(s * _RPS + i * _B, _B)])
      return carry

    lax.fori_loop(0, _ZCH, zcp, 0)

    @pl.when(have_deg)
    def _():
      one16 = jnp.ones((16,), jnp.float32)

      def fill1(i, carry):
        ones_v[pl.ds(i * 16, 16)] = one16
        return carry

      lax.fori_loop(0, 128 // 16, fill1, 0)

      def fill0(i, carry):
        dstage_v[pl.ds(i * 16, 16)] = zero16
        return carry

      lax.fori_loop(0, _DST // 16, fill0, 0)

      @pl.when(s < _N // _DST)
      def _():
        pltpu.sync_copy(dstage_v, dacc_sh.at[pl.ds(s * _DST, _DST)])

    plsc.subcore_barrier()

    def issue(g, buf, sem):
      pltpu.async_copy(table_hbm.at[src_v.at[g]], buf, sem)

    def drain_scatter(g, buf, sem):
      pltpu.make_async_copy(table_hbm.at[src_v.at[g]], buf, sem).wait()
      pltpu.sync_copy(buf, acc_sh.at[dst_v.at[g]], add=True)

    issue(0, rows_a, sem_a)

    def body(i, carry):
      g = 2 * i
      issue(g + 1, rows_b, sem_b)
      drain_scatter(g, rows_a, sem_a)
      issue(g + 2, rows_a, sem_a)
      drain_scatter(g + 1, rows_b, sem_b)
      return carry

    lax.fori_loop(0, _CH // 2 - 1, body, 0)
    issue(_CH - 1, rows_b, sem_b)
    drain_scatter(_CH - 2, rows_a, sem_a)
    drain_scatter(_CH - 1, rows_b, sem_b)

    @pl.when(have_deg)
    def _():
      def dbody(g, carry):
        pltpu.sync_copy(ones_v.at[pl.ds(0, _B)], dacc_sh.at[dst_v.at[g]],
                        add=True)
        return carry

      lax.fori_loop(0, _CH, dbody, 0)

    plsc.subcore_barrier()

    pltpu.sync_copy(acc_sh.at[pl.ds(s * _RPS, _RPS)],
                    out_hbm.at[c, pl.ds(s * _RPS, _RPS)])

    @pl.when(jnp.logical_and(have_deg, s < _N // _DST))
    def _():
      pltpu.sync_copy(dacc_sh.at[pl.ds(s * _DST, _DST)],
                      deg_hbm.at[c, pl.ds(s * _DST, _DST)])

  return k


def _rows(d):
  return pl.BlockSpec((_BM, d), lambda i: (i, 0))


def _part(d):
  return pl.BlockSpec((_NC, _BM, d), lambda i: (0, i, 0))


def _full(r, c):
  return pl.BlockSpec((r, c), lambda i: (0, 0))


def _out(d):
  return jax.ShapeDtypeStruct((_N, d), jnp.float32)


def _mm_body(x_ref, w_ref, o_ref):
  o_ref[...] = jnp.dot(x_ref[...], w_ref[...], preferred_element_type=jnp.float32)


def _tc_mm(x, w):
  fin, fout = w.shape
  return pl.pallas_call(
      _mm_body,
      grid=(_GRID,),
      in_specs=[_rows(fin), _full(fin, fout)],
      out_specs=_rows(fout),
      out_shape=_out(fout),
  )(x, w)


def _l0_body(degp, p, x, wr, b, wl1, h1_o, y1_o, inv_o):
  deg = degp[0] + degp[1]
  inv = 1.0 / jnp.maximum(deg, 1.0)
  agg = (p[0] + p[1]) * inv
  h1 = jnp.maximum(
      agg + jnp.dot(x[...], wr[...], preferred_element_type=jnp.float32) + b[...], 0.0)
  h1_o[...] = h1
  y1_o[...] = jnp.dot(h1, wl1[...], preferred_element_type=jnp.float32)
  inv_o[...] = inv


def _l1_body(p, inv, h, wr, b, o):
  agg = (p[0] + p[1]) * inv[...]
  o[...] = jnp.maximum(
      agg + jnp.dot(h[...], wr[...], preferred_element_type=jnp.float32) + b[...], 0.0)


def _l2_body(p, inv, h, wl, wr, b, oa, ob):
  agg = (p[0] + p[1]) * inv[...]
  h3 = jnp.maximum(
      jnp.dot(agg, wl[...], preferred_element_type=jnp.float32)
      + jnp.dot(h[...], wr[...], preferred_element_type=jnp.float32) + b[...], 0.0)
  oa[...] = h3[:, :64]
  ob[...] = h3[:, 64:]


def _l3_body(pa, pb, inv, ha, hb, wla, wlb, wra, wrb, b, oa, ob, oc, od):
  iv = inv[...]
  aa = (pa[0] + pa[1]) * iv
  ab = (pb[0] + pb[1]) * iv
  h4 = jnp.maximum(
      jnp.dot(aa, wla[...], preferred_element_type=jnp.float32)
      + jnp.dot(ab, wlb[...], preferred_element_type=jnp.float32)
      + jnp.dot(ha[...], wra[...], preferred_element_type=jnp.float32)
      + jnp.dot(hb[...], wrb[...], preferred_element_type=jnp.float32)
      + b[...], 0.0)
  oa[...] = h4[:, :64]
  ob[...] = h4[:, 64:128]
  oc[...] = h4[:, 128:192]
  od[...] = h4[:, 192:]


def _l4_body(pa, pb, pc, pd, inv, ha, hb, hc, hd,
             wla, wlb, wlc, wld, wra, wrb, wrc, wrd, b, wout, bout, o):
  iv = inv[...]
  h5 = (jnp.dot((pa[0] + pa[1]) * iv, wla[...], preferred_element_type=jnp.float32)
        + jnp.dot((pb[0] + pb[1]) * iv, wlb[...], preferred_element_type=jnp.float32)
        + jnp.dot((pc[0] + pc[1]) * iv, wlc[...], preferred_element_type=jnp.float32)
        + jnp.dot((pd[0] + pd[1]) * iv, wld[...], preferred_element_type=jnp.float32)
        + jnp.dot(ha[...], wra[...], preferred_element_type=jnp.float32)
        + jnp.dot(hb[...], wrb[...], preferred_element_type=jnp.float32)
        + jnp.dot(hc[...], wrc[...], preferred_element_type=jnp.float32)
        + jnp.dot(hd[...], wrd[...], preferred_element_type=jnp.float32)
        + b[...])
  h5 = jnp.maximum(h5, 0.0)
  logits = jnp.dot(h5, wout[...], preferred_element_type=jnp.float32) + bout[...]
  m = jnp.max(logits, axis=-1, keepdims=True)
  e = jnp.exp(logits - m)
  o[...] = e / jnp.sum(e, axis=-1, keepdims=True)


def kernel(x, edge_index, Wl0, Wr0, b0, Wl1, Wr1, b1, Wl2, Wr2, b2,
           Wl3, Wr3, b3, Wl4, Wr4, b4, Wout, bout):
  src2 = edge_index[0].reshape(_E // _B, _B)
  dst2 = edge_index[1].reshape(_E // _B, _B)
  b0r, b1r, b2r, b3r, b4r = (v.reshape(1, -1) for v in (b0, b1, b2, b3, b4))
  boutr = bout.reshape(1, -1)

  f1 = jnp.ones((16,), jnp.int32)
  f0 = jnp.zeros((16,), jnp.int32)
  y0 = _tc_mm(x, Wl0)
  p0, degp = _sc_agg(64)(src2, dst2, y0, f1)
  degp = degp.reshape(_NC, _N, 1)
  h1, y1, inv = pl.pallas_call(
      _l0_body,
      grid=(_GRID,),
      in_specs=[_part(1), _part(64), _rows(128), _full(128, 64),
                _full(1, 64), _full(64, 64)],
      out_specs=[_rows(64), _rows(64), _rows(1)],
      out_shape=[_out(64), _out(64), _out(1)],
  )(degp, p0, x, Wr0, b0r, Wl1)

  p1, _ = _sc_agg(64)(src2, dst2, y1, f0)
  h2 = pl.pallas_call(
      _l1_body,
      grid=(_GRID,),
      in_specs=[_part(64), _rows(1), _rows(64), _full(64, 64), _full(1, 64)],
      out_specs=_rows(64),
      out_shape=_out(64),
  )(p1, inv, h1, Wr1, b1r)

  p2, _ = _sc_agg(64)(src2, dst2, h2, f0)
  h3a, h3b = pl.pallas_call(
      _l2_body,
      grid=(_GRID,),
      in_specs=[_part(64), _rows(1), _rows(64), _full(64, 128),
                _full(64, 128), _full(1, 128)],
      out_specs=[_rows(64), _rows(64)],
      out_shape=[_out(64), _out(64)],
  )(p2, inv, h2, Wl2, Wr2, b2r)

  p3a, _ = _sc_agg(64)(src2, dst2, h3a, f0)
  p3b, _ = _sc_agg(64)(src2, dst2, h3b, f0)
  h4 = pl.pallas_call(
      _l3_body,
      grid=(_GRID,),
      in_specs=[_part(64), _part(64), _rows(1), _rows(64), _rows(64),
                _full(64, 256), _full(64, 256), _full(64, 256),
                _full(64, 256), _full(1, 256)],
      out_specs=[_rows(64)] * 4,
      out_shape=[_out(64)] * 4,
  )(p3a, p3b, inv, h3a, h3b, Wl3[:64], Wl3[64:], Wr3[:64], Wr3[64:], b3r)

  p4 = [_sc_agg(64)(src2, dst2, hq, f0)[0] for hq in h4]
  out = pl.pallas_call(
      _l4_body,
      grid=(_GRID,),
      in_specs=[_part(64)] * 4 + [_rows(1)] + [_rows(64)] * 4
               + [_full(64, 512)] * 8 + [_full(1, 512), _full(512, 4),
                                         _full(1, 4)],
      out_specs=_rows(4),
      out_shape=jax.ShapeDtypeStruct((_N, 4), jnp.float32),
  )(p4[0], p4[1], p4[2], p4[3], inv, h4[0], h4[1], h4[2], h4[3],
    Wl4[:64], Wl4[64:128], Wl4[128:192], Wl4[192:],
    Wr4[:64], Wr4[64:128], Wr4[128:192], Wr4[192:],
    b4r, Wout, boutr)
  return out


# speedup vs baseline: 10.1466x; 1.0786x over previous
import functools

import jax
import jax.numpy as jnp
from jax import lax
from jax.experimental import pallas as pl
from jax.experimental.pallas import tpu as pltpu
from jax.experimental.pallas import tpu_sc as plsc

_N = 10000
_E = 320000
_B = 125
_NC, _NS = 2, 16
_TILES = _NC * _NS
_EPT = _E // _TILES
_CH = _EPT // _B
_RPS = _N // _NS
_ZCH = _RPS // _B
_BM = 1000
_GRID = _N // _BM


_DST = 2000


@functools.lru_cache(maxsize=None)
def _sc_agg(W):
  mesh = plsc.VectorSubcoreMesh(
      core_axis_name="c", subcore_axis_name="s", num_cores=_NC, num_subcores=_NS)

  out_type = (jax.ShapeDtypeStruct((_NC, _N, W), jnp.float32),
              jax.ShapeDtypeStruct((_NC, _N), jnp.float32))
  scratch = [
      pltpu.VMEM((_CH, _B), jnp.int32),
      pltpu.VMEM((_CH, _B), jnp.int32),
      pltpu.VMEM((4, _B, W), jnp.float32),
      pltpu.VMEM((_B, W), jnp.float32),
      pltpu.VMEM_SHARED((_N, W), jnp.float32),
      pltpu.SemaphoreType.DMA,
      pltpu.SemaphoreType.DMA,
      pltpu.SemaphoreType.DMA,
      pltpu.SemaphoreType.DMA,
      pltpu.SemaphoreType.DMA,
      pltpu.SemaphoreType.DMA,
      pltpu.SemaphoreType.DMA,
      pltpu.SemaphoreType.DMA,
      pltpu.VMEM((16,), jnp.int32),
      pltpu.VMEM((128,), jnp.float32),
      pltpu.VMEM((_DST,), jnp.float32),
      pltpu.VMEM_SHARED((_N,), jnp.float32),
  ]

  @functools.partial(
      pl.kernel,
      out_type=out_type,
      mesh=mesh,
      scratch_types=scratch,
      compiler_params=pltpu.CompilerParams(
          use_tc_tiling_on_sc=False, needs_layout_passes=False),
  )
  def k(src_hbm, dst_hbm, table_hbm, dflag_hbm, *rest):
    (out_hbm, deg_hbm, src_v, dst_v, rows_v, stage_v, acc_sh,
     g0, g1, g2, g3, s0, s1, s2, s3,
     dflag_v, ones_v, dstage_v, dacc_sh) = rest
    gsem = (g0, g1, g2, g3)
    ssem = (s0, s1, s2, s3)
    c = lax.axis_index("c")
    s = lax.axis_index("s")
    wid = s * _NC + c
    pltpu.sync_copy(src_hbm.at[pl.ds(wid * _CH, _CH)], src_v)
    pltpu.sync_copy(dst_hbm.at[pl.ds(wid * _CH, _CH)], dst_v)
    pltpu.sync_copy(dflag_hbm, dflag_v)
    have_deg = jnp.sum(dflag_v[...]) > 0

    zero16 = jnp.zeros((16,), jnp.float32)

    def zrow(i, carry):
      for j in range(W // 16):
        stage_v[i, pl.ds(j * 16, 16)] = zero16
      return carry

    lax.fori_loop(0, _B, zrow, 0)

    def zcp(i, carry):
      pltpu.sync_copy(stage_v, acc_sh.at[pl.ds(s * _RPS + i * _B, _B)])
      return carry

    lax.fori_loop(0, _ZCH, zcp, 0)

    @pl.when(have_deg)
    def _():
      one16 = jnp.ones((16,), jnp.float32)

      def fill1(i, carry):
        ones_v[pl.ds(i * 16, 16)] = one16
        return carry

      lax.fori_loop(0, 128 // 16, fill1, 0)

      def fill0(i, carry):
        dstage_v[pl.ds(i * 16, 16)] = zero16
        return carry

      lax.fori_loop(0, _DST // 16, fill0, 0)

      @pl.when(s < _N // _DST)
      def _():
        pltpu.sync_copy(dstage_v, dacc_sh.at[pl.ds(s * _DST, _DST)])

    plsc.subcore_barrier()

    def issue_gather(g, j):
      pltpu.async_copy(table_hbm.at[src_v.at[g]], rows_v.at[j], gsem[j])

    def wait_gather(g, j):
      pltpu.make_async_copy(
          table_hbm.at[src_v.at[g]], rows_v.at[j], gsem[j]).wait()

    def start_scatter(g, j):
      pltpu.async_copy(rows_v.at[j], acc_sh.at[dst_v.at[g]], ssem[j],
                       add=True)

    def wait_scatter(g, j):
      pltpu.make_async_copy(rows_v.at[j], acc_sh.at[dst_v.at[g]],
                            ssem[j]).wait()

    for j in range(4):
      issue_gather(j, j)

    def body(i, carry):
      g = 4 * i
      for j in range(4):
        wait_gather(g + j, j)
        start_scatter(g + j, j)
      for j in range(4):
        wait_scatter(g + j, j)
        issue_gather(g + 4 + j, j)
      return carry

    lax.fori_loop(0, _CH // 4 - 1, body, 0)
    gl = _CH - 4
    for j in range(4):
      wait_gather(gl + j, j)
      start_scatter(gl + j, j)
    for j in range(4):
      wait_scatter(gl + j, j)

    @pl.when(have_deg)
    def _():
      def dbody(g, carry):
        pltpu.sync_copy(ones_v.at[pl.ds(0, _B)], dacc_sh.at[dst_v.at[g]],
                        add=True)
        return carry

      lax.fori_loop(0, _CH, dbody, 0)

    plsc.subcore_barrier()

    pltpu.sync_copy(acc_sh.at[pl.ds(s * _RPS, _RPS)],
                    out_hbm.at[c, pl.ds(s * _RPS, _RPS)])

    @pl.when(jnp.logical_and(have_deg, s < _N // _DST))
    def _():
      pltpu.sync_copy(dacc_sh.at[pl.ds(s * _DST, _DST)],
                      deg_hbm.at[c, pl.ds(s * _DST, _DST)])

  return k


def _rows(d):
  return pl.BlockSpec((_BM, d), lambda i: (i, 0))


def _part(d):
  return pl.BlockSpec((_NC, _BM, d), lambda i: (0, i, 0))


def _full(r, c):
  return pl.BlockSpec((r, c), lambda i: (0, 0))


def _out(d):
  return jax.ShapeDtypeStruct((_N, d), jnp.float32)


def _mm_body(x_ref, w_ref, o_ref):
  o_ref[...] = jnp.dot(x_ref[...], w_ref[...], preferred_element_type=jnp.float32)


def _tc_mm(x, w):
  fin, fout = w.shape
  return pl.pallas_call(
      _mm_body,
      grid=(_GRID,),
      in_specs=[_rows(fin), _full(fin, fout)],
      out_specs=_rows(fout),
      out_shape=_out(fout),
  )(x, w)


def _l0_body(degp, p, x, wr, b, wl1, h1_o, y1_o, inv_o):
  deg = degp[0] + degp[1]
  inv = 1.0 / jnp.maximum(deg, 1.0)
  agg = (p[0] + p[1]) * inv
  h1 = jnp.maximum(
      agg + jnp.dot(x[...], wr[...], preferred_element_type=jnp.float32) + b[...], 0.0)
  h1_o[...] = h1
  y1_o[...] = jnp.dot(h1, wl1[...], preferred_element_type=jnp.float32)
  inv_o[...] = inv


def _l1_body(p, inv, h, wr, b, o):
  agg = (p[0] + p[1]) * inv[...]
  o[...] = jnp.maximum(
      agg + jnp.dot(h[...], wr[...], preferred_element_type=jnp.float32) + b[...], 0.0)


def _l2_body(p, inv, h, wl, wr, b, oa, ob):
  agg = (p[0] + p[1]) * inv[...]
  h3 = jnp.maximum(
      jnp.dot(agg, wl[...], preferred_element_type=jnp.float32)
      + jnp.dot(h[...], wr[...], preferred_element_type=jnp.float32) + b[...], 0.0)
  oa[...] = h3[:, :64]
  ob[...] = h3[:, 64:]


def _l3_body(pa, pb, inv, ha, hb, wla, wlb, wra, wrb, b, oa, ob, oc, od):
  iv = inv[...]
  aa = (pa[0] + pa[1]) * iv
  ab = (pb[0] + pb[1]) * iv
  h4 = jnp.maximum(
      jnp.dot(aa, wla[...], preferred_element_type=jnp.float32)
      + jnp.dot(ab, wlb[...], preferred_element_type=jnp.float32)
      + jnp.dot(ha[...], wra[...], preferred_element_type=jnp.float32)
      + jnp.dot(hb[...], wrb[...], preferred_element_type=jnp.float32)
      + b[...], 0.0)
  oa[...] = h4[:, :64]
  ob[...] = h4[:, 64:128]
  oc[...] = h4[:, 128:192]
  od[...] = h4[:, 192:]


def _l4_body(pa, pb, pc, pd, inv, ha, hb, hc, hd,
             wla, wlb, wlc, wld, wra, wrb, wrc, wrd, b, wout, bout, o):
  iv = inv[...]
  h5 = (jnp.dot((pa[0] + pa[1]) * iv, wla[...], preferred_element_type=jnp.float32)
        + jnp.dot((pb[0] + pb[1]) * iv, wlb[...], preferred_element_type=jnp.float32)
        + jnp.dot((pc[0] + pc[1]) * iv, wlc[...], preferred_element_type=jnp.float32)
        + jnp.dot((pd[0] + pd[1]) * iv, wld[...], preferred_element_type=jnp.float32)
        + jnp.dot(ha[...], wra[...], preferred_element_type=jnp.float32)
        + jnp.dot(hb[...], wrb[...], preferred_element_type=jnp.float32)
        + jnp.dot(hc[...], wrc[...], preferred_element_type=jnp.float32)
        + jnp.dot(hd[...], wrd[...], preferred_element_type=jnp.float32)
        + b[...])
  h5 = jnp.maximum(h5, 0.0)
  logits = jnp.dot(h5, wout[...], preferred_element_type=jnp.float32) + bout[...]
  m = jnp.max(logits, axis=-1, keepdims=True)
  e = jnp.exp(logits - m)
  o[...] = e / jnp.sum(e, axis=-1, keepdims=True)


def kernel(x, edge_index, Wl0, Wr0, b0, Wl1, Wr1, b1, Wl2, Wr2, b2,
           Wl3, Wr3, b3, Wl4, Wr4, b4, Wout, bout):
  src2 = edge_index[0].reshape(_E // _B, _B)
  dst2 = edge_index[1].reshape(_E // _B, _B)
  b0r, b1r, b2r, b3r, b4r = (v.reshape(1, -1) for v in (b0, b1, b2, b3, b4))
  boutr = bout.reshape(1, -1)

  f1 = jnp.ones((16,), jnp.int32)
  f0 = jnp.zeros((16,), jnp.int32)
  y0 = _tc_mm(x, Wl0)
  p0, degp = _sc_agg(64)(src2, dst2, y0, f1)
  degp = degp.reshape(_NC, _N, 1)
  h1, y1, inv = pl.pallas_call(
      _l0_body,
      grid=(_GRID,),
      in_specs=[_part(1), _part(64), _rows(128), _full(128, 64),
                _full(1, 64), _full(64, 64)],
      out_specs=[_rows(64), _rows(64), _rows(1)],
      out_shape=[_out(64), _out(64), _out(1)],
  )(degp, p0, x, Wr0, b0r, Wl1)

  p1, _ = _sc_agg(64)(src2, dst2, y1, f0)
  h2 = pl.pallas_call(
      _l1_body,
      grid=(_GRID,),
      in_specs=[_part(64), _rows(1), _rows(64), _full(64, 64), _full(1, 64)],
      out_specs=_rows(64),
      out_shape=_out(64),
  )(p1, inv, h1, Wr1, b1r)

  p2, _ = _sc_agg(64)(src2, dst2, h2, f0)
  h3a, h3b = pl.pallas_call(
      _l2_body,
      grid=(_GRID,),
      in_specs=[_part(64), _rows(1), _rows(64), _full(64, 128),
                _full(64, 128), _full(1, 128)],
      out_specs=[_rows(64), _rows(64)],
      out_shape=[_out(64), _out(64)],
  )(p2, inv, h2, Wl2, Wr2, b2r)

  p3a, _ = _sc_agg(64)(src2, dst2, h3a, f0)
  p3b, _ = _sc_agg(64)(src2, dst2, h3b, f0)
  h4 = pl.pallas_call(
      _l3_body,
      grid=(_GRID,),
      in_specs=[_part(64), _part(64), _rows(1), _rows(64), _rows(64),
                _full(64, 256), _full(64, 256), _full(64, 256),
                _full(64, 256), _full(1, 256)],
      out_specs=[_rows(64)] * 4,
      out_shape=[_out(64)] * 4,
  )(p3a, p3b, inv, h3a, h3b, Wl3[:64], Wl3[64:], Wr3[:64], Wr3[64:], b3r)

  p4 = [_sc_agg(64)(src2, dst2, hq, f0)[0] for hq in h4]
  out = pl.pallas_call(
      _l4_body,
      grid=(_GRID,),
      in_specs=[_part(64)] * 4 + [_rows(1)] + [_rows(64)] * 4
               + [_full(64, 512)] * 8 + [_full(1, 512), _full(512, 4),
                                         _full(1, 4)],
      out_specs=_rows(4),
      out_shape=jax.ShapeDtypeStruct((_N, 4), jnp.float32),
  )(p4[0], p4[1], p4[2], p4[3], inv, h4[0], h4[1], h4[2], h4[3],
    Wl4[:64], Wl4[64:128], Wl4[128:192], Wl4[192:],
    Wr4[:64], Wr4[64:128], Wr4[128:192], Wr4[192:],
    b4r, Wout, boutr)
  return out
